# baseline (speedup 1.0000x reference)
import jax
import jax.numpy as jnp
from jax import lax
from jax.experimental import pallas as pl
from jax.experimental.pallas import tpu as pltpu
from jax.experimental.pallas import tpu_sc as plsc

N = 20
D = 64
B = 16384
OUTW = 3 * D
PW = 2 * D
SPLIT = 524288
TS = 999936
TAIL = 64
TOFF = TS - SPLIT
PKH = SPLIT + TAIL
NC = 2
NS = 16
NW = NC * NS
TBLK = 512
TGRID = (PKH + TBLK - 1) // TBLK
BCLAMP = TS // TBLK
BPW = B // NW
CB = 32
NCHUNK = BPW // CB
IPC = CB * N
GSZ = 128
NG = IPC // GSZ
NVD = D // 16


def _tc_transpose(a_ref, b_ref, o_ref):
    r = lax.broadcasted_iota(jnp.int32, (D, D), 0)
    c = lax.broadcasted_iota(jnp.int32, (D, D), 1)
    eye = jnp.where(r == c, 1.0, 0.0).astype(jnp.float32)
    dn = (((0,), (0,)), ((), ()))
    at = lax.dot_general(a_ref[...], eye, dn,
                         preferred_element_type=jnp.float32)
    bt = lax.dot_general(b_ref[...], eye, dn,
                         preferred_element_type=jnp.float32)
    o_ref[...] = jnp.concatenate([at, bt], axis=1)


def _sc_body(mem_idx_hbm, user_hbm, user_pk, item_pk, wb_hbm, out_hbm,
             idx_v, gidx_v, uidx_v, ugidx_v, items_v, urows_v, outb_v, wb_v,
             sem):
    wid = lax.axis_index("s") * NC + lax.axis_index("c")
    base = wid * BPW

    pltpu.sync_copy(mem_idx_hbm.at[pl.ds(base * N, BPW * N)], idx_v)
    pltpu.sync_copy(user_hbm.at[pl.ds(base, BPW)], uidx_v)
    pltpu.sync_copy(wb_hbm, wb_v)

    def _pack(v):
        return jnp.where(v >= TS, v - TOFF,
                         v - jnp.where(v >= SPLIT, SPLIT, 0))

    def shift_body(i, carry):
        v = idx_v[pl.ds(i * 16, 16)]
        gidx_v[pl.ds(i * 16, 16)] = _pack(v)
        return carry

    lax.fori_loop(0, BPW * N // 16, shift_body, 0)

    def ushift_body(i, carry):
        v = uidx_v[pl.ds(i * 16, 16)]
        ugidx_v[pl.ds(i * 16, 16)] = _pack(v)
        return carry

    lax.fori_loop(0, BPW // 16, ushift_body, 0)

    wv = [wb_v[n, :] for n in range(N)]
    bias = wb_v[N, :]

    def chunk(j, carry):
        cps = [pltpu.async_copy(item_pk.at[gidx_v.at[pl.ds(j * IPC + g * GSZ, GSZ)]],
                                items_v.at[pl.ds(g * GSZ, GSZ)], sem)
               for g in range(NG)]
        cps.append(pltpu.async_copy(user_pk.at[ugidx_v.at[pl.ds(j * CB, CB)]],
                                    urows_v, sem))
        for c in cps:
            c.wait()

        def bbody(k, c2):
            uvv = uidx_v[pl.ds(j * CB + k * 16, 16)]
            duv = jnp.where(uvv >= SPLIT, D, 0)
            for bi in range(16):
                b = k * 16 + bi
                row0 = b * N
                i0 = idx_v[pl.ds(j * IPC + row0, 16)]
                i1 = idx_v[pl.ds(j * IPC + row0 + 4, 16)]
                iv0 = jnp.where(i0 >= SPLIT, D, 0)
                iv1 = jnp.where(i1 >= SPLIT, D, 0)
                du = duv[bi]
                di = [iv0[n] for n in range(16)] + [iv1[n - 4] for n in range(16, N)]
                for d in range(NVD):
                    u = urows_v[b, pl.ds(du + d * 16, 16)]
                    acc = bias
                    for n in range(N):
                        acc = acc + wv[n] * items_v[row0 + n,
                                                    pl.ds(di[n] + d * 16, 16)]
                    outb_v[b, pl.ds(d * 16, 16)] = u
                    outb_v[b, pl.ds(D + d * 16, 16)] = u * acc
                    outb_v[b, pl.ds(2 * D + d * 16, 16)] = acc
            return c2

        lax.fori_loop(0, CB // 16, bbody, 0)
        pltpu.sync_copy(outb_v, out_hbm.at[pl.ds(base + j * CB, CB)])
        return carry

    lax.fori_loop(0, NCHUNK, chunk, 0)


def _mesh():
    return plsc.VectorSubcoreMesh(core_axis_name="c", subcore_axis_name="s",
                                  num_cores=NC, num_subcores=NS)


def _transpose_call(t):
    return pl.pallas_call(
        _tc_transpose,
        grid=(TGRID,),
        in_specs=[
            pl.BlockSpec((D, TBLK), lambda i: (0, i)),
            pl.BlockSpec((D, TBLK),
                         lambda i: (0, jnp.minimum(SPLIT // TBLK + i, BCLAMP))),
        ],
        out_specs=pl.BlockSpec((TBLK, PW), lambda i: (i, 0)),
        out_shape=jax.ShapeDtypeStruct((PKH, PW), jnp.float32),
    )(t, t)


@jax.jit
def _run(user, mem_flat, tu, ti, wb):
    user_pk = _transpose_call(tu)
    item_pk = _transpose_call(ti)

    gather = pl.kernel(
        _sc_body,
        out_type=jax.ShapeDtypeStruct((B, OUTW), jnp.float32),
        mesh=_mesh(),
        scratch_types=[
            pltpu.VMEM((BPW * N,), jnp.int32),
            pltpu.VMEM((BPW * N,), jnp.int32),
            pltpu.VMEM((BPW,), jnp.int32),
            pltpu.VMEM((BPW,), jnp.int32),
            pltpu.VMEM((IPC, PW), jnp.float32),
            pltpu.VMEM((CB, PW), jnp.float32),
            pltpu.VMEM((CB, OUTW), jnp.float32),
            pltpu.VMEM((N + 1, 16), jnp.float32),
            pltpu.SemaphoreType.DMA,
        ],
    )
    return gather(mem_flat, user, user_pk, item_pk, wb)


def kernel(user, memory, user_table, item_table, conv_w, conv_b):
    w = conv_w.reshape(N)
    wb = jnp.broadcast_to(jnp.concatenate([w, conv_b]).reshape(N + 1, 1),
                          (N + 1, 16)).astype(jnp.float32)
    mem_flat = memory.astype(jnp.int32).reshape(B * N)
    user = user.astype(jnp.int32)
    return _run(user, mem_flat, user_table.T, item_table.T, wb)


# speedup vs baseline: 1.4699x; 1.4699x over previous
import jax
import jax.numpy as jnp
from jax import lax
from jax.experimental import pallas as pl
from jax.experimental.pallas import tpu as pltpu
from jax.experimental.pallas import tpu_sc as plsc

N = 20
D = 64
B = 16384
OUTW = 3 * D
PW = 2 * D
SPLIT = 524288
TS = 999936
TAIL = 64
TBLK = 1024
TAILP = SPLIT + (TS % TBLK)
TOFF = TS - TAILP
PKH = TAILP + TAIL
TGRID = (PKH + TBLK - 1) // TBLK
BCLAMP = TS // TBLK
NC = 2
NS = 16
NW = NC * NS
BPW = B // NW
CB = 32
NCHUNK = BPW // CB
IPC = CB * N
GSZ = 128
NG = IPC // GSZ
NVD = D // 16


def _tc_transpose(a_ref, b_ref, o_ref):
    o_ref[...] = jnp.concatenate([a_ref[...].T, b_ref[...].T], axis=1)


def _sc_body(mem_idx_hbm, user_hbm, user_pk, item_pk, wb_hbm, out_hbm,
             idx_v, gidx_v, uidx_v, ugidx_v, items_v, urows_v, outb_v, wb_v,
             sem):
    wid = lax.axis_index("s") * NC + lax.axis_index("c")
    base = wid * BPW

    pltpu.sync_copy(mem_idx_hbm.at[pl.ds(base * N, BPW * N)], idx_v)
    pltpu.sync_copy(user_hbm.at[pl.ds(base, BPW)], uidx_v)
    pltpu.sync_copy(wb_hbm, wb_v)

    def _pack(v):
        return jnp.where(v >= TS, v - TOFF,
                         v - jnp.where(v >= SPLIT, SPLIT, 0))

    def shift_body(i, carry):
        v = idx_v[pl.ds(i * 16, 16)]
        gidx_v[pl.ds(i * 16, 16)] = _pack(v)
        return carry

    lax.fori_loop(0, BPW * N // 16, shift_body, 0)

    def ushift_body(i, carry):
        v = uidx_v[pl.ds(i * 16, 16)]
        ugidx_v[pl.ds(i * 16, 16)] = _pack(v)
        return carry

    lax.fori_loop(0, BPW // 16, ushift_body, 0)

    wv = [wb_v[n, :] for n in range(N)]
    bias = wb_v[N, :]

    def chunk(j, carry):
        cps = [pltpu.async_copy(item_pk.at[gidx_v.at[pl.ds(j * IPC + g * GSZ, GSZ)]],
                                items_v.at[pl.ds(g * GSZ, GSZ)], sem)
               for g in range(NG)]
        cps.append(pltpu.async_copy(user_pk.at[ugidx_v.at[pl.ds(j * CB, CB)]],
                                    urows_v, sem))
        for c in cps:
            c.wait()

        def bbody(k, c2):
            uvv = uidx_v[pl.ds(j * CB + k * 16, 16)]
            duv = jnp.where(uvv >= SPLIT, D, 0)
            for bi in range(16):
                b = k * 16 + bi
                row0 = b * N
                i0 = idx_v[pl.ds(j * IPC + row0, 16)]
                i1 = idx_v[pl.ds(j * IPC + row0 + 4, 16)]
                iv0 = jnp.where(i0 >= SPLIT, D, 0)
                iv1 = jnp.where(i1 >= SPLIT, D, 0)
                du = duv[bi]
                di = [iv0[n] for n in range(16)] + [iv1[n - 4] for n in range(16, N)]
                for d in range(NVD):
                    u = urows_v[b, pl.ds(du + d * 16, 16)]
                    acc = bias
                    for n in range(N):
                        acc = acc + wv[n] * items_v[row0 + n,
                                                    pl.ds(di[n] + d * 16, 16)]
                    outb_v[b, pl.ds(d * 16, 16)] = u
                    outb_v[b, pl.ds(D + d * 16, 16)] = u * acc
                    outb_v[b, pl.ds(2 * D + d * 16, 16)] = acc
            return c2

        lax.fori_loop(0, CB // 16, bbody, 0)
        pltpu.sync_copy(outb_v, out_hbm.at[pl.ds(base + j * CB, CB)])
        return carry

    lax.fori_loop(0, NCHUNK, chunk, 0)


def _mesh():
    return plsc.VectorSubcoreMesh(core_axis_name="c", subcore_axis_name="s",
                                  num_cores=NC, num_subcores=NS)


def _transpose_call(t):
    return pl.pallas_call(
        _tc_transpose,
        grid=(TGRID,),
        in_specs=[
            pl.BlockSpec((D, TBLK), lambda i: (0, i)),
            pl.BlockSpec((D, TBLK),
                         lambda i: (0, jnp.minimum(SPLIT // TBLK + i, BCLAMP))),
        ],
        out_specs=pl.BlockSpec((TBLK, PW), lambda i: (i, 0)),
        out_shape=jax.ShapeDtypeStruct((PKH, PW), jnp.float32),
    )(t, t)


@jax.jit
def _run(user, mem_flat, tu, ti, wb):
    user_pk = _transpose_call(tu)
    item_pk = _transpose_call(ti)

    gather = pl.kernel(
        _sc_body,
        out_type=jax.ShapeDtypeStruct((B, OUTW), jnp.float32),
        mesh=_mesh(),
        scratch_types=[
            pltpu.VMEM((BPW * N,), jnp.int32),
            pltpu.VMEM((BPW * N,), jnp.int32),
            pltpu.VMEM((BPW,), jnp.int32),
            pltpu.VMEM((BPW,), jnp.int32),
            pltpu.VMEM((IPC, PW), jnp.float32),
            pltpu.VMEM((CB, PW), jnp.float32),
            pltpu.VMEM((CB, OUTW), jnp.float32),
            pltpu.VMEM((N + 1, 16), jnp.float32),
            pltpu.SemaphoreType.DMA,
        ],
    )
    return gather(mem_flat, user, user_pk, item_pk, wb)


def kernel(user, memory, user_table, item_table, conv_w, conv_b):
    w = conv_w.reshape(N)
    wb = jnp.broadcast_to(jnp.concatenate([w, conv_b]).reshape(N + 1, 1),
                          (N + 1, 16)).astype(jnp.float32)
    mem_flat = memory.astype(jnp.int32).reshape(B * N)
    user = user.astype(jnp.int32)
    return _run(user, mem_flat, user_table.T, item_table.T, wb)


# speedup vs baseline: 1.8610x; 1.2661x over previous
import jax
import jax.numpy as jnp
from jax import lax
from jax.experimental import pallas as pl
from jax.experimental.pallas import tpu as pltpu
from jax.experimental.pallas import tpu_sc as plsc

N = 20
D = 64
B = 16384
OUTW = 3 * D
PW = 2 * D
SPLIT = 524288
TS = 999936
TAIL = 64
TBLK = 2048
TAILP = SPLIT + (TS % TBLK)
TOFF = TS - TAILP
PKH = TAILP + TAIL
TGRID = (PKH + TBLK - 1) // TBLK
BCLAMP = TS // TBLK
NC = 2
NS = 16
NW = NC * NS
BPW = B // NW
CB = 32
NCHUNK = BPW // CB
IPC = CB * N
GSZ = 128
NG = IPC // GSZ
NVD = D // 16


def _tc_transpose(a_ref, b_ref, o_ref):
    o_ref[...] = jnp.concatenate([a_ref[...].T, b_ref[...].T], axis=1)


def _sc_body(mem_idx_hbm, user_hbm, user_pk, item_pk, wb_hbm, out_hbm,
             idx_v, gidx_v, uidx_v, ugidx_v, items_v, urows_v, outb_v, wb_v,
             sem):
    wid = lax.axis_index("s") * NC + lax.axis_index("c")
    base = wid * BPW

    pltpu.sync_copy(mem_idx_hbm.at[pl.ds(base * N, BPW * N)], idx_v)
    pltpu.sync_copy(user_hbm.at[pl.ds(base, BPW)], uidx_v)
    pltpu.sync_copy(wb_hbm, wb_v)

    def _pack(v):
        return jnp.where(v >= TS, v - TOFF,
                         v - jnp.where(v >= SPLIT, SPLIT, 0))

    def shift_body(i, carry):
        v = idx_v[pl.ds(i * 16, 16)]
        gidx_v[pl.ds(i * 16, 16)] = _pack(v)
        return carry

    lax.fori_loop(0, BPW * N // 16, shift_body, 0)

    def ushift_body(i, carry):
        v = uidx_v[pl.ds(i * 16, 16)]
        ugidx_v[pl.ds(i * 16, 16)] = _pack(v)
        return carry

    lax.fori_loop(0, BPW // 16, ushift_body, 0)

    wv = [wb_v[n, :] for n in range(N)]
    bias = wb_v[N, :]

    def chunk(j, carry):
        cps = [pltpu.async_copy(item_pk.at[gidx_v.at[pl.ds(j * IPC + g * GSZ, GSZ)]],
                                items_v.at[pl.ds(g * GSZ, GSZ)], sem)
               for g in range(NG)]
        cps.append(pltpu.async_copy(user_pk.at[ugidx_v.at[pl.ds(j * CB, CB)]],
                                    urows_v, sem))
        for c in cps:
            c.wait()

        def bbody(k, c2):
            uvv = uidx_v[pl.ds(j * CB + k * 16, 16)]
            duv = jnp.where(uvv >= SPLIT, D, 0)
            for bi in range(16):
                b = k * 16 + bi
                row0 = b * N
                i0 = idx_v[pl.ds(j * IPC + row0, 16)]
                i1 = idx_v[pl.ds(j * IPC + row0 + 4, 16)]
                iv0 = jnp.where(i0 >= SPLIT, D, 0)
                iv1 = jnp.where(i1 >= SPLIT, D, 0)
                du = duv[bi]
                di = [iv0[n] for n in range(16)] + [iv1[n - 4] for n in range(16, N)]
                for d in range(NVD):
                    u = urows_v[b, pl.ds(du + d * 16, 16)]
                    acc = bias
                    for n in range(N):
                        acc = acc + wv[n] * items_v[row0 + n,
                                                    pl.ds(di[n] + d * 16, 16)]
                    outb_v[b, pl.ds(d * 16, 16)] = u
                    outb_v[b, pl.ds(D + d * 16, 16)] = u * acc
                    outb_v[b, pl.ds(2 * D + d * 16, 16)] = acc
            return c2

        lax.fori_loop(0, CB // 16, bbody, 0)
        pltpu.sync_copy(outb_v, out_hbm.at[pl.ds(base + j * CB, CB)])
        return carry

    lax.fori_loop(0, NCHUNK, chunk, 0)


def _mesh():
    return plsc.VectorSubcoreMesh(core_axis_name="c", subcore_axis_name="s",
                                  num_cores=NC, num_subcores=NS)


def _transpose_call(t):
    return pl.pallas_call(
        _tc_transpose,
        grid=(TGRID,),
        in_specs=[
            pl.BlockSpec((D, TBLK), lambda i: (0, i)),
            pl.BlockSpec((D, TBLK),
                         lambda i: (0, jnp.minimum(SPLIT // TBLK + i, BCLAMP))),
        ],
        out_specs=pl.BlockSpec((TBLK, PW), lambda i: (i, 0)),
        out_shape=jax.ShapeDtypeStruct((PKH, PW), jnp.float32),
    )(t, t)


@jax.jit
def _run(user, mem_flat, tu, ti, wb):
    user_pk = _transpose_call(tu)
    item_pk = _transpose_call(ti)

    gather = pl.kernel(
        _sc_body,
        out_type=jax.ShapeDtypeStruct((B, OUTW), jnp.float32),
        mesh=_mesh(),
        scratch_types=[
            pltpu.VMEM((BPW * N,), jnp.int32),
            pltpu.VMEM((BPW * N,), jnp.int32),
            pltpu.VMEM((BPW,), jnp.int32),
            pltpu.VMEM((BPW,), jnp.int32),
            pltpu.VMEM((IPC, PW), jnp.float32),
            pltpu.VMEM((CB, PW), jnp.float32),
            pltpu.VMEM((CB, OUTW), jnp.float32),
            pltpu.VMEM((N + 1, 16), jnp.float32),
            pltpu.SemaphoreType.DMA,
        ],
    )
    return gather(mem_flat, user, user_pk, item_pk, wb)


def kernel(user, memory, user_table, item_table, conv_w, conv_b):
    w = conv_w.reshape(N)
    wb = jnp.broadcast_to(jnp.concatenate([w, conv_b]).reshape(N + 1, 1),
                          (N + 1, 16)).astype(jnp.float32)
    mem_flat = memory.astype(jnp.int32).reshape(B * N)
    user = user.astype(jnp.int32)
    return _run(user, mem_flat, user_table.T, item_table.T, wb)


# speedup vs baseline: 2.1653x; 1.1635x over previous
import jax
import jax.numpy as jnp
from jax import lax
from jax.experimental import pallas as pl
from jax.experimental.pallas import tpu as pltpu
from jax.experimental.pallas import tpu_sc as plsc

N = 20
D = 64
B = 16384
OUTW = 3 * D
PW = 2 * D
SPLIT = 524288
TS = 999936
TAIL = 64
TBLK = 4096
TAILP = SPLIT + (TS % TBLK)
TOFF = TS - TAILP
PKH = TAILP + TAIL
TGRID = (PKH + TBLK - 1) // TBLK
BCLAMP = TS // TBLK
NC = 2
NS = 16
NW = NC * NS
BPW = B // NW
CB = 32
NCHUNK = BPW // CB
IPC = CB * N
GSZ = 128
NG = IPC // GSZ
NVD = D // 16


def _tc_transpose(a_ref, b_ref, o_ref):
    o_ref[...] = jnp.concatenate([a_ref[...].T, b_ref[...].T], axis=1)


def _sc_body(mem_idx_hbm, user_hbm, user_pk, item_pk, wb_hbm, out_hbm,
             idx_v, gidx_v, uidx_v, ugidx_v, items_v, urows_v, outb_v, wb_v,
             sem):
    wid = lax.axis_index("s") * NC + lax.axis_index("c")
    base = wid * BPW

    pltpu.sync_copy(mem_idx_hbm.at[pl.ds(base * N, BPW * N)], idx_v)
    pltpu.sync_copy(user_hbm.at[pl.ds(base, BPW)], uidx_v)
    pltpu.sync_copy(wb_hbm, wb_v)

    def _pack(v):
        return jnp.where(v >= TS, v - TOFF,
                         v - jnp.where(v >= SPLIT, SPLIT, 0))

    def shift_body(i, carry):
        v = idx_v[pl.ds(i * 16, 16)]
        gidx_v[pl.ds(i * 16, 16)] = _pack(v)
        return carry

    lax.fori_loop(0, BPW * N // 16, shift_body, 0)

    def ushift_body(i, carry):
        v = uidx_v[pl.ds(i * 16, 16)]
        ugidx_v[pl.ds(i * 16, 16)] = _pack(v)
        return carry

    lax.fori_loop(0, BPW // 16, ushift_body, 0)

    wv = [wb_v[n, :] for n in range(N)]
    bias = wb_v[N, :]

    def chunk(j, carry):
        cps = [pltpu.async_copy(item_pk.at[gidx_v.at[pl.ds(j * IPC + g * GSZ, GSZ)]],
                                items_v.at[pl.ds(g * GSZ, GSZ)], sem)
               for g in range(NG)]
        cps.append(pltpu.async_copy(user_pk.at[ugidx_v.at[pl.ds(j * CB, CB)]],
                                    urows_v, sem))
        for c in cps:
            c.wait()

        def bbody(k, c2):
            uvv = uidx_v[pl.ds(j * CB + k * 16, 16)]
            duv = jnp.where(uvv >= SPLIT, D, 0)
            for bi in range(16):
                b = k * 16 + bi
                row0 = b * N
                i0 = idx_v[pl.ds(j * IPC + row0, 16)]
                i1 = idx_v[pl.ds(j * IPC + row0 + 4, 16)]
                iv0 = jnp.where(i0 >= SPLIT, D, 0)
                iv1 = jnp.where(i1 >= SPLIT, D, 0)
                du = duv[bi]
                di = [iv0[n] for n in range(16)] + [iv1[n - 4] for n in range(16, N)]
                for d in range(NVD):
                    u = urows_v[b, pl.ds(du + d * 16, 16)]
                    acc = bias
                    for n in range(N):
                        acc = acc + wv[n] * items_v[row0 + n,
                                                    pl.ds(di[n] + d * 16, 16)]
                    outb_v[b, pl.ds(d * 16, 16)] = u
                    outb_v[b, pl.ds(D + d * 16, 16)] = u * acc
                    outb_v[b, pl.ds(2 * D + d * 16, 16)] = acc
            return c2

        lax.fori_loop(0, CB // 16, bbody, 0)
        pltpu.sync_copy(outb_v, out_hbm.at[pl.ds(base + j * CB, CB)])
        return carry

    lax.fori_loop(0, NCHUNK, chunk, 0)


def _mesh():
    return plsc.VectorSubcoreMesh(core_axis_name="c", subcore_axis_name="s",
                                  num_cores=NC, num_subcores=NS)


def _transpose_call(t):
    return pl.pallas_call(
        _tc_transpose,
        grid=(TGRID,),
        in_specs=[
            pl.BlockSpec((D, TBLK), lambda i: (0, i)),
            pl.BlockSpec((D, TBLK),
                         lambda i: (0, jnp.minimum(SPLIT // TBLK + i, BCLAMP))),
        ],
        out_specs=pl.BlockSpec((TBLK, PW), lambda i: (i, 0)),
        out_shape=jax.ShapeDtypeStruct((PKH, PW), jnp.float32),
    )(t, t)


@jax.jit
def _run(user, mem_flat, tu, ti, wb):
    user_pk = _transpose_call(tu)
    item_pk = _transpose_call(ti)

    gather = pl.kernel(
        _sc_body,
        out_type=jax.ShapeDtypeStruct((B, OUTW), jnp.float32),
        mesh=_mesh(),
        scratch_types=[
            pltpu.VMEM((BPW * N,), jnp.int32),
            pltpu.VMEM((BPW * N,), jnp.int32),
            pltpu.VMEM((BPW,), jnp.int32),
            pltpu.VMEM((BPW,), jnp.int32),
            pltpu.VMEM((IPC, PW), jnp.float32),
            pltpu.VMEM((CB, PW), jnp.float32),
            pltpu.VMEM((CB, OUTW), jnp.float32),
            pltpu.VMEM((N + 1, 16), jnp.float32),
            pltpu.SemaphoreType.DMA,
        ],
    )
    return gather(mem_flat, user, user_pk, item_pk, wb)


def kernel(user, memory, user_table, item_table, conv_w, conv_b):
    w = conv_w.reshape(N)
    wb = jnp.broadcast_to(jnp.concatenate([w, conv_b]).reshape(N + 1, 1),
                          (N + 1, 16)).astype(jnp.float32)
    mem_flat = memory.astype(jnp.int32).reshape(B * N)
    user = user.astype(jnp.int32)
    return _run(user, mem_flat, user_table.T, item_table.T, wb)


# speedup vs baseline: 2.3599x; 1.0898x over previous
import jax
import jax.numpy as jnp
from jax import lax
from jax.experimental import pallas as pl
from jax.experimental.pallas import tpu as pltpu
from jax.experimental.pallas import tpu_sc as plsc

N = 20
D = 64
B = 16384
OUTW = 3 * D
PW = 2 * D
SPLIT = 524288
TS = 999936
TAIL = 64
TBLK = 8192
TAILP = SPLIT + (TS % TBLK)
TOFF = TS - TAILP
PKH = TAILP + TAIL
TGRID = (PKH + TBLK - 1) // TBLK
BCLAMP = TS // TBLK
NC = 2
NS = 16
NW = NC * NS
BPW = B // NW
CB = 32
NCHUNK = BPW // CB
IPC = CB * N
GSZ = 128
NG = IPC // GSZ
NVD = D // 16


def _tc_transpose(a_ref, b_ref, o_ref):
    o_ref[...] = jnp.concatenate([a_ref[...].T, b_ref[...].T], axis=1)


def _sc_body(mem_idx_hbm, user_hbm, user_pk, item_pk, wb_hbm, out_hbm,
             idx_v, gidx_v, uidx_v, ugidx_v, items_v, urows_v, outb_v, wb_v,
             sem):
    wid = lax.axis_index("s") * NC + lax.axis_index("c")
    base = wid * BPW

    pltpu.sync_copy(mem_idx_hbm.at[pl.ds(base * N, BPW * N)], idx_v)
    pltpu.sync_copy(user_hbm.at[pl.ds(base, BPW)], uidx_v)
    pltpu.sync_copy(wb_hbm, wb_v)

    def _pack(v):
        return jnp.where(v >= TS, v - TOFF,
                         v - jnp.where(v >= SPLIT, SPLIT, 0))

    def shift_body(i, carry):
        v = idx_v[pl.ds(i * 16, 16)]
        gidx_v[pl.ds(i * 16, 16)] = _pack(v)
        return carry

    lax.fori_loop(0, BPW * N // 16, shift_body, 0)

    def ushift_body(i, carry):
        v = uidx_v[pl.ds(i * 16, 16)]
        ugidx_v[pl.ds(i * 16, 16)] = _pack(v)
        return carry

    lax.fori_loop(0, BPW // 16, ushift_body, 0)

    wv = [wb_v[n, :] for n in range(N)]
    bias = wb_v[N, :]

    def chunk(j, carry):
        cps = [pltpu.async_copy(item_pk.at[gidx_v.at[pl.ds(j * IPC + g * GSZ, GSZ)]],
                                items_v.at[pl.ds(g * GSZ, GSZ)], sem)
               for g in range(NG)]
        cps.append(pltpu.async_copy(user_pk.at[ugidx_v.at[pl.ds(j * CB, CB)]],
                                    urows_v, sem))
        for c in cps:
            c.wait()

        def bbody(k, c2):
            uvv = uidx_v[pl.ds(j * CB + k * 16, 16)]
            duv = jnp.where(uvv >= SPLIT, D, 0)
            for bi in range(16):
                b = k * 16 + bi
                row0 = b * N
                i0 = idx_v[pl.ds(j * IPC + row0, 16)]
                i1 = idx_v[pl.ds(j * IPC + row0 + 4, 16)]
                iv0 = jnp.where(i0 >= SPLIT, D, 0)
                iv1 = jnp.where(i1 >= SPLIT, D, 0)
                du = duv[bi]
                di = [iv0[n] for n in range(16)] + [iv1[n - 4] for n in range(16, N)]
                for d in range(NVD):
                    u = urows_v[b, pl.ds(du + d * 16, 16)]
                    acc = bias
                    for n in range(N):
                        acc = acc + wv[n] * items_v[row0 + n,
                                                    pl.ds(di[n] + d * 16, 16)]
                    outb_v[b, pl.ds(d * 16, 16)] = u
                    outb_v[b, pl.ds(D + d * 16, 16)] = u * acc
                    outb_v[b, pl.ds(2 * D + d * 16, 16)] = acc
            return c2

        lax.fori_loop(0, CB // 16, bbody, 0)
        pltpu.sync_copy(outb_v, out_hbm.at[pl.ds(base + j * CB, CB)])
        return carry

    lax.fori_loop(0, NCHUNK, chunk, 0)


def _mesh():
    return plsc.VectorSubcoreMesh(core_axis_name="c", subcore_axis_name="s",
                                  num_cores=NC, num_subcores=NS)


def _transpose_call(t):
    return pl.pallas_call(
        _tc_transpose,
        grid=(TGRID,),
        in_specs=[
            pl.BlockSpec((D, TBLK), lambda i: (0, i)),
            pl.BlockSpec((D, TBLK),
                         lambda i: (0, jnp.minimum(SPLIT // TBLK + i, BCLAMP))),
        ],
        out_specs=pl.BlockSpec((TBLK, PW), lambda i: (i, 0)),
        out_shape=jax.ShapeDtypeStruct((PKH, PW), jnp.float32),
    )(t, t)


@jax.jit
def _run(user, mem_flat, tu, ti, wb):
    user_pk = _transpose_call(tu)
    item_pk = _transpose_call(ti)

    gather = pl.kernel(
        _sc_body,
        out_type=jax.ShapeDtypeStruct((B, OUTW), jnp.float32),
        mesh=_mesh(),
        scratch_types=[
            pltpu.VMEM((BPW * N,), jnp.int32),
            pltpu.VMEM((BPW * N,), jnp.int32),
            pltpu.VMEM((BPW,), jnp.int32),
            pltpu.VMEM((BPW,), jnp.int32),
            pltpu.VMEM((IPC, PW), jnp.float32),
            pltpu.VMEM((CB, PW), jnp.float32),
            pltpu.VMEM((CB, OUTW), jnp.float32),
            pltpu.VMEM((N + 1, 16), jnp.float32),
            pltpu.SemaphoreType.DMA,
        ],
    )
    return gather(mem_flat, user, user_pk, item_pk, wb)


def kernel(user, memory, user_table, item_table, conv_w, conv_b):
    w = conv_w.reshape(N)
    wb = jnp.broadcast_to(jnp.concatenate([w, conv_b]).reshape(N + 1, 1),
                          (N + 1, 16)).astype(jnp.float32)
    mem_flat = memory.astype(jnp.int32).reshape(B * N)
    user = user.astype(jnp.int32)
    return _run(user, mem_flat, user_table.T, item_table.T, wb)


# speedup vs baseline: 2.4477x; 1.0372x over previous
import jax
import jax.numpy as jnp
from jax import lax
from jax.experimental import pallas as pl
from jax.experimental.pallas import tpu as pltpu
from jax.experimental.pallas import tpu_sc as plsc

N = 20
D = 64
B = 16384
OUTW = 3 * D
PW = 2 * D
SPLIT = 524288
TS = 999936
TAIL = 64
TBLK = 16384
TAILP = SPLIT + (TS % TBLK)
TOFF = TS - TAILP
PKH = TAILP + TAIL
TGRID = (PKH + TBLK - 1) // TBLK
BCLAMP = TS // TBLK
NC = 2
NS = 16
NW = NC * NS
BPW = B // NW
CB = 32
NCHUNK = BPW // CB
IPC = CB * N
GSZ = 128
NG = IPC // GSZ
NVD = D // 16


def _tc_transpose(a_ref, b_ref, o_ref):
    o_ref[...] = jnp.concatenate([a_ref[...].T, b_ref[...].T], axis=1)


def _sc_body(mem_idx_hbm, user_hbm, user_pk, item_pk, wb_hbm, out_hbm,
             idx_v, gidx_v, uidx_v, ugidx_v, items_v, urows_v, outb_v, wb_v,
             sem):
    wid = lax.axis_index("s") * NC + lax.axis_index("c")
    base = wid * BPW

    pltpu.sync_copy(mem_idx_hbm.at[pl.ds(base * N, BPW * N)], idx_v)
    pltpu.sync_copy(user_hbm.at[pl.ds(base, BPW)], uidx_v)
    pltpu.sync_copy(wb_hbm, wb_v)

    def _pack(v):
        return jnp.where(v >= TS, v - TOFF,
                         v - jnp.where(v >= SPLIT, SPLIT, 0))

    def shift_body(i, carry):
        v = idx_v[pl.ds(i * 16, 16)]
        gidx_v[pl.ds(i * 16, 16)] = _pack(v)
        return carry

    lax.fori_loop(0, BPW * N // 16, shift_body, 0)

    def ushift_body(i, carry):
        v = uidx_v[pl.ds(i * 16, 16)]
        ugidx_v[pl.ds(i * 16, 16)] = _pack(v)
        return carry

    lax.fori_loop(0, BPW // 16, ushift_body, 0)

    wv = [wb_v[n, :] for n in range(N)]
    bias = wb_v[N, :]

    def chunk(j, carry):
        cps = [pltpu.async_copy(item_pk.at[gidx_v.at[pl.ds(j * IPC + g * GSZ, GSZ)]],
                                items_v.at[pl.ds(g * GSZ, GSZ)], sem)
               for g in range(NG)]
        cps.append(pltpu.async_copy(user_pk.at[ugidx_v.at[pl.ds(j * CB, CB)]],
                                    urows_v, sem))
        for c in cps:
            c.wait()

        def bbody(k, c2):
            uvv = uidx_v[pl.ds(j * CB + k * 16, 16)]
            duv = jnp.where(uvv >= SPLIT, D, 0)
            for bi in range(16):
                b = k * 16 + bi
                row0 = b * N
                i0 = idx_v[pl.ds(j * IPC + row0, 16)]
                i1 = idx_v[pl.ds(j * IPC + row0 + 4, 16)]
                iv0 = jnp.where(i0 >= SPLIT, D, 0)
                iv1 = jnp.where(i1 >= SPLIT, D, 0)
                du = duv[bi]
                di = [iv0[n] for n in range(16)] + [iv1[n - 4] for n in range(16, N)]
                for d in range(NVD):
                    u = urows_v[b, pl.ds(du + d * 16, 16)]
                    acc = bias
                    for n in range(N):
                        acc = acc + wv[n] * items_v[row0 + n,
                                                    pl.ds(di[n] + d * 16, 16)]
                    outb_v[b, pl.ds(d * 16, 16)] = u
                    outb_v[b, pl.ds(D + d * 16, 16)] = u * acc
                    outb_v[b, pl.ds(2 * D + d * 16, 16)] = acc
            return c2

        lax.fori_loop(0, CB // 16, bbody, 0)
        pltpu.sync_copy(outb_v, out_hbm.at[pl.ds(base + j * CB, CB)])
        return carry

    lax.fori_loop(0, NCHUNK, chunk, 0)


def _mesh():
    return plsc.VectorSubcoreMesh(core_axis_name="c", subcore_axis_name="s",
                                  num_cores=NC, num_subcores=NS)


def _transpose_call(t):
    return pl.pallas_call(
        _tc_transpose,
        grid=(TGRID,),
        in_specs=[
            pl.BlockSpec((D, TBLK), lambda i: (0, i)),
            pl.BlockSpec((D, TBLK),
                         lambda i: (0, jnp.minimum(SPLIT // TBLK + i, BCLAMP))),
        ],
        out_specs=pl.BlockSpec((TBLK, PW), lambda i: (i, 0)),
        out_shape=jax.ShapeDtypeStruct((PKH, PW), jnp.float32),
    )(t, t)


@jax.jit
def _run(user, mem_flat, tu, ti, wb):
    user_pk = _transpose_call(tu)
    item_pk = _transpose_call(ti)

    gather = pl.kernel(
        _sc_body,
        out_type=jax.ShapeDtypeStruct((B, OUTW), jnp.float32),
        mesh=_mesh(),
        scratch_types=[
            pltpu.VMEM((BPW * N,), jnp.int32),
            pltpu.VMEM((BPW * N,), jnp.int32),
            pltpu.VMEM((BPW,), jnp.int32),
            pltpu.VMEM((BPW,), jnp.int32),
            pltpu.VMEM((IPC, PW), jnp.float32),
            pltpu.VMEM((CB, PW), jnp.float32),
            pltpu.VMEM((CB, OUTW), jnp.float32),
            pltpu.VMEM((N + 1, 16), jnp.float32),
            pltpu.SemaphoreType.DMA,
        ],
    )
    return gather(mem_flat, user, user_pk, item_pk, wb)


def kernel(user, memory, user_table, item_table, conv_w, conv_b):
    w = conv_w.reshape(N)
    wb = jnp.broadcast_to(jnp.concatenate([w, conv_b]).reshape(N + 1, 1),
                          (N + 1, 16)).astype(jnp.float32)
    mem_flat = memory.astype(jnp.int32).reshape(B * N)
    user = user.astype(jnp.int32)
    return _run(user, mem_flat, user_table.T, item_table.T, wb)
